# Initial kernel scaffold; baseline (speedup 1.0000x reference)
import functools

import jax
import jax.numpy as jnp
from jax import lax
from jax.experimental import pallas as pl
from jax.experimental.pallas import tpu as pltpu
from jax.experimental.pallas import tpu_sc as plsc

NQ = 8
V = 100000
D = 32
B = 4096
T = 50

NR = 128
GL = 128
GPC = NR * NQ // GL
ROWS = B * T


def _sc_call():
    info_nc, info_ns = 2, 16
    try:
        info = plsc.get_sparse_core_info()
        info_nc, info_ns = info.num_cores, info.num_subcores
    except Exception:
        pass
    NW = info_nc * info_ns
    rows_per_w = ROWS // NW
    nch = rows_per_w // NR

    mesh = plsc.VectorSubcoreMesh(core_axis_name="c", subcore_axis_name="s")

    @functools.partial(
        pl.kernel,
        mesh=mesh,
        out_type=jax.ShapeDtypeStruct((ROWS, D), jnp.float32),
        scratch_types=[
            pltpu.VMEM((GPC, GL), jnp.int32),
            pltpu.VMEM((GPC, GL), jnp.int32),
            pltpu.VMEM((NR * NQ, D), jnp.float32),
            pltpu.VMEM((NR * NQ, D), jnp.float32),
            pltpu.VMEM((NR, D), jnp.float32),
            pltpu.VMEM((NR, D), jnp.float32),
            pltpu.VMEM((NQ, 16), jnp.float32),
            pltpu.SemaphoreType.DMA,
            pltpu.SemaphoreType.DMA,
            pltpu.SemaphoreType.DMA,
            pltpu.SemaphoreType.DMA,
        ],
    )
    def k(idx_hbm, table_hbm, w_hbm, out_hbm,
          idx0, idx1, rows0, rows1, outb0, outb1, wv,
          gsem0, gsem1, osem0, osem1):
        wid = lax.axis_index("s") * info_nc + lax.axis_index("c")
        idxs = (idx0, idx1)
        rows = (rows0, rows1)
        outs = (outb0, outb1)
        gsems = (gsem0, gsem1)
        osems = (osem0, osem1)

        pltpu.sync_copy(w_hbm, wv)
        w = [wv[i, :] for i in range(NQ)]

        def issue(c, buf):
            r0 = (wid * nch + c) * GPC
            pltpu.sync_copy(idx_hbm.at[pl.ds(r0, GPC)], idxs[buf])
            for j in range(GPC):
                pltpu.async_copy(
                    table_hbm.at[idxs[buf].at[j]],
                    rows[buf].at[pl.ds(j * GL, GL)],
                    gsems[buf],
                )

        def wait_gathers(buf):
            pltpu.make_async_copy(
                table_hbm.at[pl.ds(0, NR * NQ)], rows[buf], gsems[buf]
            ).wait()

        def out_slice(c):
            return out_hbm.at[pl.ds(wid * rows_per_w + c * NR, NR)]

        def compute(buf):
            rref = rows[buf]
            oref = outs[buf]

            def row_body(j, carry):
                base = j * NQ
                a0 = w[0] * rref[base, pl.ds(0, 16)]
                a1 = w[0] * rref[base, pl.ds(16, 16)]
                for i in range(1, NQ):
                    a0 = a0 + w[i] * rref[base + i, pl.ds(0, 16)]
                    a1 = a1 + w[i] * rref[base + i, pl.ds(16, 16)]
                oref[j, pl.ds(0, 16)] = a0
                oref[j, pl.ds(16, 16)] = a1
                return carry

            lax.fori_loop(0, NR, row_body, 0)

        def stage(c, buf):
            @pl.when(c + 1 < nch)
            def _():
                issue(c + 1, (buf + 1) % 2)

            wait_gathers(buf)

            @pl.when(c >= 2)
            def _():
                pltpu.make_async_copy(outs[buf], out_slice(c), osems[buf]).wait()

            compute(buf)
            pltpu.async_copy(outs[buf], out_slice(c), osems[buf])

        issue(0, 0)

        def outer(g, carry):
            stage(2 * g, 0)
            stage(2 * g + 1, 1)
            return carry

        lax.fori_loop(0, nch // 2, outer, 0)

        pltpu.make_async_copy(outs[0], out_slice(nch - 2), osems[0]).wait()
        pltpu.make_async_copy(outs[1], out_slice(nch - 1), osems[1]).wait()

    return k


def kernel(tokens, tables, weights):
    flat_table = tables.reshape(NQ * V, D)
    offs = (jnp.arange(NQ, dtype=jnp.int32) * V)
    idx = (tokens.reshape(B, NQ, T).transpose(0, 2, 1) + offs[None, None, :])
    idx = idx.reshape(ROWS * NQ // GL, GL)
    w16 = jnp.broadcast_to(weights.astype(jnp.float32)[:, None], (NQ, 16))
    out = _sc_call()(idx, flat_table, w16)
    return out.reshape(B, T, D)


# speedup vs baseline: 8.6467x; 8.6467x over previous
import functools

import jax
import jax.numpy as jnp
from jax import lax
from jax.experimental import pallas as pl
from jax.experimental.pallas import tpu as pltpu
from jax.experimental.pallas import tpu_sc as plsc

NQ = 8
V = 100000
D = 32
B = 4096
T = 50

NR = 128
GL = 128
GPC = NR * NQ // GL
ROWS = B * T


def _sc_call():
    info_nc, info_ns = 2, 16
    try:
        info = plsc.get_sparse_core_info()
        info_nc, info_ns = info.num_cores, info.num_subcores
    except Exception:
        pass
    NW = info_nc * info_ns
    rows_per_w = ROWS // NW
    nch = rows_per_w // NR

    mesh = plsc.VectorSubcoreMesh(core_axis_name="c", subcore_axis_name="s")

    @functools.partial(
        pl.kernel,
        mesh=mesh,
        out_type=jax.ShapeDtypeStruct((ROWS, D), jnp.float32),
        compiler_params=pltpu.CompilerParams(use_tc_tiling_on_sc=False),
        scratch_types=[
            pltpu.VMEM((GPC, GL), jnp.int32),
            pltpu.VMEM((GPC, GL), jnp.int32),
            pltpu.VMEM((NR * NQ, D), jnp.float32),
            pltpu.VMEM((NR * NQ, D), jnp.float32),
            pltpu.VMEM((NR, D), jnp.float32),
            pltpu.VMEM((NR, D), jnp.float32),
            pltpu.VMEM((NQ, 16), jnp.float32),
            pltpu.SemaphoreType.DMA,
            pltpu.SemaphoreType.DMA,
            pltpu.SemaphoreType.DMA,
            pltpu.SemaphoreType.DMA,
        ],
    )
    def k(idx_hbm, table_hbm, w_hbm, out_hbm,
          idx0, idx1, rows0, rows1, outb0, outb1, wv,
          gsem0, gsem1, osem0, osem1):
        wid = lax.axis_index("s") * info_nc + lax.axis_index("c")
        idxs = (idx0, idx1)
        rows = (rows0, rows1)
        outs = (outb0, outb1)
        gsems = (gsem0, gsem1)
        osems = (osem0, osem1)

        pltpu.sync_copy(w_hbm, wv)
        w = [wv[i, :] for i in range(NQ)]

        def issue(c, buf):
            r0 = (wid * nch + c) * GPC
            pltpu.sync_copy(idx_hbm.at[pl.ds(r0, GPC)], idxs[buf])
            for j in range(GPC):
                pltpu.async_copy(
                    table_hbm.at[idxs[buf].at[j]],
                    rows[buf].at[pl.ds(j * GL, GL)],
                    gsems[buf],
                )

        def wait_gathers(buf):
            pltpu.make_async_copy(
                table_hbm.at[pl.ds(0, NR * NQ)], rows[buf], gsems[buf]
            ).wait()

        def out_slice(c):
            return out_hbm.at[pl.ds(wid * rows_per_w + c * NR, NR)]

        def compute(buf):
            rref = rows[buf]
            oref = outs[buf]

            def row_body(j, carry):
                base = j * NQ
                a0 = w[0] * rref[base, pl.ds(0, 16)]
                a1 = w[0] * rref[base, pl.ds(16, 16)]
                for i in range(1, NQ):
                    a0 = a0 + w[i] * rref[base + i, pl.ds(0, 16)]
                    a1 = a1 + w[i] * rref[base + i, pl.ds(16, 16)]
                oref[j, pl.ds(0, 16)] = a0
                oref[j, pl.ds(16, 16)] = a1
                return carry

            lax.fori_loop(0, NR, row_body, 0)

        def stage(c, buf):
            @pl.when(c + 1 < nch)
            def _():
                issue(c + 1, (buf + 1) % 2)

            wait_gathers(buf)

            @pl.when(c >= 2)
            def _():
                pltpu.make_async_copy(outs[buf], out_slice(c), osems[buf]).wait()

            compute(buf)
            pltpu.async_copy(outs[buf], out_slice(c), osems[buf])

        issue(0, 0)

        def outer(g, carry):
            stage(2 * g, 0)
            stage(2 * g + 1, 1)
            return carry

        lax.fori_loop(0, nch // 2, outer, 0)

        pltpu.make_async_copy(outs[0], out_slice(nch - 2), osems[0]).wait()
        pltpu.make_async_copy(outs[1], out_slice(nch - 1), osems[1]).wait()

    return k


def kernel(tokens, tables, weights):
    flat_table = tables.reshape(NQ * V, D)
    offs = (jnp.arange(NQ, dtype=jnp.int32) * V)
    idx = (tokens.reshape(B, NQ, T).transpose(0, 2, 1) + offs[None, None, :])
    idx = idx.reshape(ROWS * NQ // GL, GL)
    w16 = jnp.broadcast_to(weights.astype(jnp.float32)[:, None], (NQ, 16))
    out = _sc_call()(idx, flat_table, w16)
    return out.reshape(B, T, D)


# speedup vs baseline: 10.3856x; 1.2011x over previous
import functools

import jax
import jax.numpy as jnp
from jax import lax
from jax.experimental import pallas as pl
from jax.experimental.pallas import tpu as pltpu
from jax.experimental.pallas import tpu_sc as plsc

NQ = 8
V = 100000
D = 32
B = 4096
T = 50

NB = 2
GL = 80
GPC = NB * T * NQ // GL
ROWS = B * T


def _sc_call():
    info_nc, info_ns = 2, 16
    try:
        info = plsc.get_sparse_core_info()
        info_nc, info_ns = info.num_cores, info.num_subcores
    except Exception:
        pass
    NW = info_nc * info_ns
    b_per_w = B // NW
    nch = b_per_w // NB

    mesh = plsc.VectorSubcoreMesh(core_axis_name="c", subcore_axis_name="s")

    @functools.partial(
        pl.kernel,
        mesh=mesh,
        out_type=jax.ShapeDtypeStruct((B, T, D), jnp.float32),
        compiler_params=pltpu.CompilerParams(use_tc_tiling_on_sc=False),
        scratch_types=[
            pltpu.VMEM((GPC, GL), jnp.int32),
            pltpu.VMEM((GPC, GL), jnp.int32),
            pltpu.VMEM((NB * T * NQ, D), jnp.float32),
            pltpu.VMEM((NB * T * NQ, D), jnp.float32),
            pltpu.VMEM((NB, T, D), jnp.float32),
            pltpu.VMEM((NB, T, D), jnp.float32),
            pltpu.VMEM((NQ, 16), jnp.float32),
            pltpu.SemaphoreType.DMA,
            pltpu.SemaphoreType.DMA,
            pltpu.SemaphoreType.DMA,
            pltpu.SemaphoreType.DMA,
        ],
    )
    def k(idx_hbm, table_hbm, w_hbm, out_hbm,
          idx0, idx1, rows0, rows1, outb0, outb1, wv,
          gsem0, gsem1, osem0, osem1):
        wid = lax.axis_index("s") * info_nc + lax.axis_index("c")
        idxs = (idx0, idx1)
        rows = (rows0, rows1)
        outs = (outb0, outb1)
        gsems = (gsem0, gsem1)
        osems = (osem0, osem1)

        pltpu.sync_copy(w_hbm, wv)
        w = [wv[q, :] for q in range(NQ)]

        def issue(c, buf):
            r0 = (wid * nch + c) * GPC
            pltpu.sync_copy(idx_hbm.at[pl.ds(r0, GPC)], idxs[buf])
            for g in range(GPC):
                pltpu.async_copy(
                    table_hbm.at[idxs[buf].at[g]],
                    rows[buf].at[pl.ds(g * GL, GL)],
                    gsems[buf],
                )

        def wait_gathers(buf):
            pltpu.make_async_copy(
                table_hbm.at[pl.ds(0, NB * T * NQ)], rows[buf], gsems[buf]
            ).wait()

        def out_slice(c):
            return out_hbm.at[pl.ds(wid * b_per_w + c * NB, NB)]

        def compute(buf):
            rref = rows[buf]
            oref = outs[buf]

            for b in range(NB):
                def row_body(t, carry, b=b):
                    base = (b * T + t) * NQ
                    a0 = w[0] * rref[base, pl.ds(0, 16)]
                    a1 = w[0] * rref[base, pl.ds(16, 16)]
                    for q in range(1, NQ):
                        a0 = a0 + w[q] * rref[base + q, pl.ds(0, 16)]
                        a1 = a1 + w[q] * rref[base + q, pl.ds(16, 16)]
                    oref[b, t, pl.ds(0, 16)] = a0
                    oref[b, t, pl.ds(16, 16)] = a1
                    return carry

                lax.fori_loop(0, T, row_body, 0)

        def stage(c, buf):
            @pl.when(c + 1 < nch)
            def _():
                issue(c + 1, (buf + 1) % 2)

            wait_gathers(buf)

            @pl.when(c >= 2)
            def _():
                pltpu.make_async_copy(outs[buf], out_slice(c), osems[buf]).wait()

            compute(buf)
            pltpu.async_copy(outs[buf], out_slice(c), osems[buf])

        issue(0, 0)

        def outer(g, carry):
            stage(2 * g, 0)
            stage(2 * g + 1, 1)
            return carry

        lax.fori_loop(0, nch // 2, outer, 0)

        pltpu.make_async_copy(outs[0], out_slice(nch - 2), osems[0]).wait()
        pltpu.make_async_copy(outs[1], out_slice(nch - 1), osems[1]).wait()

    return k


def kernel(tokens, tables, weights):
    flat_table = tables.reshape(NQ * V, D)
    offs = jnp.arange(NQ, dtype=jnp.int32) * V
    idx = tokens.reshape(B, NQ, T).transpose(0, 2, 1) + offs[None, None, :]
    idx = idx.reshape(ROWS * NQ // GL, GL)
    w16 = jnp.broadcast_to(weights.astype(jnp.float32)[:, None], (NQ, 16))
    return _sc_call()(idx, flat_table, w16)


# speedup vs baseline: 11.7127x; 1.1278x over previous
import functools

import jax
import jax.numpy as jnp
from jax import lax
from jax.experimental import pallas as pl
from jax.experimental.pallas import tpu as pltpu
from jax.experimental.pallas import tpu_sc as plsc

NQ = 8
V = 100000
D = 32
B = 4096
T = 50

NB = 2
GL = 80
GPC = NB * T * NQ // GL
ROWS = B * T


def _sc_call():
    info_nc, info_ns = 2, 16
    try:
        info = plsc.get_sparse_core_info()
        info_nc, info_ns = info.num_cores, info.num_subcores
    except Exception:
        pass
    NW = info_nc * info_ns
    b_per_w = B // NW
    nch = b_per_w // NB

    mesh = plsc.VectorSubcoreMesh(core_axis_name="c", subcore_axis_name="s")

    @functools.partial(
        pl.kernel,
        mesh=mesh,
        out_type=jax.ShapeDtypeStruct((B, T, D), jnp.float32),
        compiler_params=pltpu.CompilerParams(use_tc_tiling_on_sc=False),
        scratch_types=[
            pltpu.VMEM((NB, T * NQ), jnp.int32),
            pltpu.VMEM((NB, T * NQ), jnp.int32),
            pltpu.VMEM((T * NQ,), jnp.int32),
            pltpu.VMEM((GPC, GL), jnp.int32),
            pltpu.VMEM((GPC, GL), jnp.int32),
            pltpu.VMEM((NB * T * NQ, D), jnp.float32),
            pltpu.VMEM((NB * T * NQ, D), jnp.float32),
            pltpu.VMEM((NB, T, D), jnp.float32),
            pltpu.VMEM((NB, T, D), jnp.float32),
            pltpu.VMEM((NQ, 16), jnp.float32),
            pltpu.SemaphoreType.DMA,
            pltpu.SemaphoreType.DMA,
            pltpu.SemaphoreType.DMA,
            pltpu.SemaphoreType.DMA,
        ],
    )
    def k(tok_hbm, table_hbm, w_hbm, out_hbm,
          tok0, tok1, offs_v, idx0, idx1, rows0, rows1, outb0, outb1, wv,
          gsem0, gsem1, osem0, osem1):
        wid = lax.axis_index("s") * info_nc + lax.axis_index("c")
        toks = (tok0, tok1)
        idxs = (idx0, idx1)
        rows = (rows0, rows1)
        outs = (outb0, outb1)
        gsems = (gsem0, gsem1)
        osems = (osem0, osem1)

        pltpu.sync_copy(w_hbm, wv)
        w = [wv[q, :] for q in range(NQ)]

        W = T * NQ
        zero16 = lax.iota(jnp.int32, 16) * 0
        for q in range(NQ):
            for s in (0, 16, 32, T - 16):
                offs_v[pl.ds(q * T + s, 16)] = zero16 + (q * V)

        def issue(c, buf):
            b0 = wid * b_per_w + c * NB
            pltpu.sync_copy(tok_hbm.at[pl.ds(b0, NB)], toks[buf])
            for g in range(GPC):
                for kk in range(GL // 16):
                    p = g * GL + kk * 16
                    idxs[buf][g, pl.ds(kk * 16, 16)] = (
                        toks[buf][p // W, pl.ds(p % W, 16)]
                        + offs_v[pl.ds(p % W, 16)]
                    )
            for g in range(GPC):
                pltpu.async_copy(
                    table_hbm.at[idxs[buf].at[g]],
                    rows[buf].at[pl.ds(g * GL, GL)],
                    gsems[buf],
                )

        def wait_gathers(buf):
            pltpu.make_async_copy(
                table_hbm.at[pl.ds(0, NB * T * NQ)], rows[buf], gsems[buf]
            ).wait()

        def out_slice(c):
            return out_hbm.at[pl.ds(wid * b_per_w + c * NB, NB)]

        def compute(buf):
            rref = rows[buf]
            oref = outs[buf]

            for b in range(NB):
                def row_body(t, carry, b=b):
                    base = b * T * NQ + t
                    a0 = w[0] * rref[base, pl.ds(0, 16)]
                    a1 = w[0] * rref[base, pl.ds(16, 16)]
                    for q in range(1, NQ):
                        a0 = a0 + w[q] * rref[base + q * T, pl.ds(0, 16)]
                        a1 = a1 + w[q] * rref[base + q * T, pl.ds(16, 16)]
                    oref[b, t, pl.ds(0, 16)] = a0
                    oref[b, t, pl.ds(16, 16)] = a1
                    return carry

                lax.fori_loop(0, T, row_body, 0)

        def stage(c, buf):
            @pl.when(c + 1 < nch)
            def _():
                issue(c + 1, (buf + 1) % 2)

            wait_gathers(buf)

            @pl.when(c >= 2)
            def _():
                pltpu.make_async_copy(outs[buf], out_slice(c), osems[buf]).wait()

            compute(buf)
            pltpu.async_copy(outs[buf], out_slice(c), osems[buf])

        issue(0, 0)

        def outer(g, carry):
            stage(2 * g, 0)
            stage(2 * g + 1, 1)
            return carry

        lax.fori_loop(0, nch // 2, outer, 0)

        pltpu.make_async_copy(outs[0], out_slice(nch - 2), osems[0]).wait()
        pltpu.make_async_copy(outs[1], out_slice(nch - 1), osems[1]).wait()

    return k


def kernel(tokens, tables, weights):
    flat_table = tables.reshape(NQ * V, D)
    w16 = jnp.broadcast_to(weights.astype(jnp.float32)[:, None], (NQ, 16))
    return _sc_call()(tokens, flat_table, w16)


# speedup vs baseline: 11.8862x; 1.0148x over previous
import functools

import jax
import jax.numpy as jnp
from jax import lax
from jax.experimental import pallas as pl
from jax.experimental.pallas import tpu as pltpu
from jax.experimental.pallas import tpu_sc as plsc

NQ = 8
V = 100000
D = 32
B = 4096
T = 50
W = NQ * T

NBW = 128
GL = 128
GPC = NBW * NQ // GL


def _sc_call():
    info_nc, info_ns = 2, 16
    try:
        info = plsc.get_sparse_core_info()
        info_nc, info_ns = info.num_cores, info.num_subcores
    except Exception:
        pass
    NW = info_nc * info_ns
    assert B // NW == NBW

    mesh = plsc.VectorSubcoreMesh(core_axis_name="c", subcore_axis_name="s")

    @functools.partial(
        pl.kernel,
        mesh=mesh,
        out_type=jax.ShapeDtypeStruct((T * D // 8, NW, 8 * NBW), jnp.float32),
        compiler_params=pltpu.CompilerParams(
            use_tc_tiling_on_sc=False, needs_layout_passes=False),
        scratch_types=[
            pltpu.VMEM((NBW * W,), jnp.int32),
            pltpu.VMEM((GPC, GL), jnp.int32),
            pltpu.VMEM((GPC, GL), jnp.int32),
            pltpu.VMEM((NBW * NQ, D), jnp.float32),
            pltpu.VMEM((NBW * NQ, D), jnp.float32),
            pltpu.VMEM((D // 8, 1, 8 * NBW), jnp.float32),
            pltpu.VMEM((D // 8, 1, 8 * NBW), jnp.float32),
            pltpu.VMEM((NQ, 16), jnp.float32),
            pltpu.SemaphoreType.DMA,
            pltpu.SemaphoreType.DMA,
            pltpu.SemaphoreType.DMA,
            pltpu.SemaphoreType.DMA,
            pltpu.SemaphoreType.DMA,
        ],
    )
    def k(tok_hbm, table_hbm, w_hbm, out_hbm,
          tokv, idx0, idx1, rows0, rows1, outb0, outb1, wv,
          tsem, gsem0, gsem1, osem0, osem1):
        wid = lax.axis_index("s") * info_nc + lax.axis_index("c")
        idxs = (idx0, idx1)
        rows = (rows0, rows1)
        outs = (outb0, outb1)
        gsems = (gsem0, gsem1)
        osems = (osem0, osem1)

        pltpu.sync_copy(w_hbm, wv)
        w = [wv[q, :] for q in range(NQ)]
        lane = lax.iota(jnp.int32, 16)
        bstride = lane * W
        scat_q = lane * 0
        d0a = lane // 8
        d0b = d0a + 2
        zero16 = lane * 0
        d2base = (lane % 8) * NBW

        pltpu.async_copy(
            tok_hbm.at[pl.ds(wid * (NBW * W), NBW * W)], tokv, tsem
        ).wait()

        def issue(c, buf):
            for g in range(GPC):
                for q in range(NQ):
                    toks16 = plsc.load_gather(
                        tokv, [bstride + (g * 16 * W + q * T + c)]
                    )
                    plsc.store_scatter(
                        idxs[buf].at[g], [lane * NQ + q], toks16 + (q * V)
                    )
            for g in range(GPC):
                pltpu.async_copy(
                    table_hbm.at[idxs[buf].at[g]],
                    rows[buf].at[pl.ds(g * GL, GL)],
                    gsems[buf],
                )

        def wait_gathers(buf):
            pltpu.make_async_copy(
                table_hbm.at[pl.ds(0, NBW * NQ)], rows[buf], gsems[buf]
            ).wait()

        def out_slice(c):
            return out_hbm.at[pl.ds(c * (D // 8), D // 8), pl.ds(wid, 1)]

        def compute(buf):
            rref = rows[buf]
            oref = outs[buf]

            def row_body(b, carry):
                base = b * NQ
                a0 = w[0] * rref[base, pl.ds(0, 16)]
                a1 = w[0] * rref[base, pl.ds(16, 16)]
                for q in range(1, NQ):
                    a0 = a0 + w[q] * rref[base + q, pl.ds(0, 16)]
                    a1 = a1 + w[q] * rref[base + q, pl.ds(16, 16)]
                d2 = d2base + b
                plsc.store_scatter(oref, [d0a, zero16, d2], a0)
                plsc.store_scatter(oref, [d0b, zero16, d2], a1)
                return carry

            lax.fori_loop(0, NBW, row_body, 0)

        def stage(c, buf):
            @pl.when(c + 1 < T)
            def _():
                issue(c + 1, (buf + 1) % 2)

            wait_gathers(buf)

            @pl.when(c >= 2)
            def _():
                pltpu.make_async_copy(outs[buf], out_slice(c), osems[buf]).wait()

            compute(buf)
            pltpu.async_copy(outs[buf], out_slice(c), osems[buf])

        issue(0, 0)

        def outer(g, carry):
            stage(2 * g, 0)
            stage(2 * g + 1, 1)
            return carry

        lax.fori_loop(0, T // 2, outer, 0)

        pltpu.make_async_copy(outs[0], out_slice(T - 2), osems[0]).wait()
        pltpu.make_async_copy(outs[1], out_slice(T - 1), osems[1]).wait()

    return k


def kernel(tokens, tables, weights):
    flat_table = tables.reshape(NQ * V, D)
    w16 = jnp.broadcast_to(weights.astype(jnp.float32)[:, None], (NQ, 16))
    out3 = _sc_call()(tokens.reshape(B * W), flat_table, w16)
    out5 = out3.reshape(T, D // 8, 32, 8, NBW)
    return out5.transpose(2, 4, 0, 1, 3).reshape(B, T, D)
